# baseline (speedup 1.0000x reference)
import functools

import jax
import jax.numpy as jnp
from jax import lax
from jax.experimental import pallas as pl
from jax.experimental.pallas import tpu as pltpu
from jax.experimental.pallas import tpu_sc as plsc

_INTERPRET = False


def _s2d(x):
    n, h, w, c = x.shape
    x = x.reshape(n, h // 2, 2, w // 2, 2, c)
    x = jnp.transpose(x, (0, 1, 3, 2, 4, 5))
    return x.reshape(n, h // 2, w // 2, 4 * c)


def _s2d_inv(x):
    n, h, w, c4 = x.shape
    c = c4 // 4
    x = x.reshape(n, h, w, 2, 2, c)
    x = jnp.transpose(x, (0, 1, 3, 2, 4, 5))
    return x.reshape(n, 2 * h, 2 * w, c)


_S2_TAP = ((-1, 1), (0, 0), (0, 1), (1, 0))


def _s2w(w):
    o, i = w.shape[0], w.shape[1]
    out = jnp.zeros((3, 3, 4 * i, o), jnp.float32)
    for ky in range(4):
        dy, py = _S2_TAP[ky]
        for kx in range(4):
            dx, px = _S2_TAP[kx]
            g = py * 2 + px
            out = out.at[dy + 1, dx + 1, g * i:(g + 1) * i, :].set(w[:, :, ky, kx].T)
    return out


_CT_TAP = {0: ((1, 0), (3, -1)), 1: ((0, 1), (2, 0))}


def _ctw(w):
    i, o = w.shape[0], w.shape[1]
    out = jnp.zeros((3, 3, i, 4 * o), jnp.float32)
    for py in (0, 1):
        for ky, dy in _CT_TAP[py]:
            for px in (0, 1):
                for kx, dx in _CT_TAP[px]:
                    g = py * 2 + px
                    out = out.at[dy + 1, dx + 1, :, g * o:(g + 1) * o].set(w[:, :, ky, kx])
    return out


def _w3x3(w):
    return jnp.transpose(w, (2, 3, 1, 0))


def _conv(x, w, b, rows, relu_in=False, relu_out=False, pw=None, pb=None):
    n, h, wd, c = x.shape
    co = w.shape[-1]
    cf = pw.shape[-1] if pw is not None else co
    r = rows
    nr = h // r
    xp = jnp.pad(x, ((0, 0), (1, 1), (1, 1), (0, 0)))
    xs = jnp.stack([xp[:, i * r:i * r + r + 2] for i in range(nr)], axis=1)
    w3 = w.reshape(3, 3 * c, co)
    bb = b.reshape(1, co)

    ins = [xs, w3, bb]
    in_specs = [
        pl.BlockSpec((1, 1, r + 2, wd + 2, c), lambda ni, ri: (ni, ri, 0, 0, 0)),
        pl.BlockSpec((3, 3 * c, co), lambda ni, ri: (0, 0, 0)),
        pl.BlockSpec((1, co), lambda ni, ri: (0, 0)),
    ]
    if pw is not None:
        ins += [pw, pb.reshape(1, cf)]
        in_specs += [
            pl.BlockSpec((co, cf), lambda ni, ri: (0, 0)),
            pl.BlockSpec((1, cf), lambda ni, ri: (0, 0)),
        ]

    def body(*refs):
        if pw is None:
            x_ref, w_ref, b_ref, o_ref = refs
        else:
            x_ref, w_ref, b_ref, pw_ref, pb_ref, o_ref = refs
        xb = x_ref[0, 0]
        if relu_in:
            xb = jnp.maximum(xb, 0.0)
        xc = jnp.concatenate(
            [xb[:, 0:wd], xb[:, 1:wd + 1], xb[:, 2:wd + 2]], axis=-1)
        acc = None
        for dy in range(3):
            t = jnp.dot(xc[dy:dy + r].reshape(r * wd, 3 * c), w_ref[dy],
                        preferred_element_type=jnp.float32)
            acc = t if acc is None else acc + t
        acc = acc + b_ref[:]
        if relu_out or pw is not None:
            acc = jnp.maximum(acc, 0.0)
        if pw is not None:
            acc = jnp.dot(acc, pw_ref[:],
                          preferred_element_type=jnp.float32) + pb_ref[:]
        o_ref[0, 0] = acc.reshape(r, wd, cf)

    out = pl.pallas_call(
        body,
        grid=(n, nr),
        in_specs=in_specs,
        out_specs=pl.BlockSpec((1, 1, r, wd, cf), lambda ni, ri: (ni, ri, 0, 0, 0)),
        out_shape=jax.ShapeDtypeStruct((n, nr, r, wd, cf), jnp.float32),
        interpret=_INTERPRET,
    )(*ins)
    return out.reshape(n, h, wd, cf)


def _vq(zf, embed, m):
    bsz, d = zf.shape
    e = embed.shape[0]
    nb = bsz // m
    et = embed.T

    def body(z_ref, et_ref, idx_ref, ls_ref):
        z = z_ref[:]
        ze = jnp.dot(z, et_ref[:], preferred_element_type=jnp.float32)
        z2 = jnp.sum(z * z, axis=1, keepdims=True)
        e2 = jnp.sum(et_ref[:] * et_ref[:], axis=0, keepdims=True)
        dist = z2 - 2.0 * ze + e2
        dmin = jnp.min(dist, axis=1, keepdims=True)
        ii = lax.broadcasted_iota(jnp.int32, dist.shape, 1)
        idxv = jnp.min(jnp.where(dist == dmin, ii, e), axis=1, keepdims=True)
        idx_ref[:] = idxv.astype(jnp.int32)
        s = jnp.sum(dmin).reshape(1, 1)

        @pl.when(pl.program_id(0) == 0)
        def _():
            ls_ref[:] = s

        @pl.when(pl.program_id(0) != 0)
        def _():
            ls_ref[:] = ls_ref[:] + s

    idx, ls = pl.pallas_call(
        body,
        grid=(nb,),
        in_specs=[
            pl.BlockSpec((m, d), lambda i: (i, 0)),
            pl.BlockSpec((d, e), lambda i: (0, 0)),
        ],
        out_specs=[
            pl.BlockSpec((m, 1), lambda i: (i, 0)),
            pl.BlockSpec((1, 1), lambda i: (0, 0)),
        ],
        out_shape=[
            jax.ShapeDtypeStruct((bsz, 1), jnp.int32),
            jax.ShapeDtypeStruct((1, 1), jnp.float32),
        ],
        interpret=_INTERPRET,
    )(zf, et)
    return idx.reshape(bsz), ls[0, 0]


def _gather_zq(embed, idx):
    e, d = embed.shape
    bsz = idx.shape[0]
    dp = 128
    table = jnp.pad(embed, ((0, 0), (0, dp - d)))
    info = plsc.get_sparse_core_info()
    nc, ns = info.num_cores, info.num_subcores
    nw = nc * ns
    bpw = bsz // nw
    nchunk = 2
    half = bpw // nchunk
    mesh = plsc.VectorSubcoreMesh(core_axis_name="c", subcore_axis_name="s")

    @functools.partial(
        pl.kernel,
        mesh=mesh,
        out_type=jax.ShapeDtypeStruct((bsz, dp), jnp.float32),
        scratch_types=[
            pltpu.VMEM((bpw,), jnp.int32),
            pltpu.VMEM((half, dp), jnp.float32),
            pltpu.SemaphoreType.DMA,
        ],
    )
    def k(table_hbm, idx_hbm, out_hbm, idx_v, rows_v, sem):
        wid = lax.axis_index("s") * nc + lax.axis_index("c")
        base = wid * bpw
        pltpu.sync_copy(idx_hbm.at[pl.ds(base, bpw)], idx_v)
        for j in range(nchunk):
            pltpu.async_copy(
                table_hbm.at[idx_v.at[pl.ds(j * half, half)]], rows_v, sem
            ).wait()
            pltpu.sync_copy(rows_v, out_hbm.at[pl.ds(base + j * half, half)])

    return k(table, idx)[:, :d]


def kernel(x, enc_w1, enc_b1, enc_w2, enc_b2, enc_w3, enc_b3, enc_res_w1,
           enc_res_b1, enc_res_w2, enc_res_b2, embed, dec_w1, dec_b1,
           dec_res_w1, dec_res_b1, dec_res_w2, dec_res_b2, dec_w2, dec_b2,
           dec_w3, dec_b3):
    n = x.shape[0]
    a = jnp.transpose(x, (0, 2, 3, 1))
    a = _s2d(a)
    a = _conv(a, _s2w(enc_w1), enc_b1, rows=16, relu_out=True)
    a = _s2d(a)
    a = _conv(a, _s2w(enc_w2), enc_b2, rows=8, relu_out=True)
    a = _conv(a, _w3x3(enc_w3), enc_b3, rows=16)
    for i in range(enc_res_w1.shape[0]):
        a = _conv(a, _w3x3(enc_res_w1[i]), enc_res_b1[i], rows=32,
                  relu_in=True, pw=enc_res_w2[i][:, :, 0, 0].T,
                  pb=enc_res_b2[i])

    zf = a.reshape(-1, embed.shape[1])
    idx, lsum = _vq(zf, embed, m=1024)
    loss = 1.25 * lsum / zf.size
    zq = _gather_zq(embed, idx)

    t = zq.reshape(n, 96, 96, embed.shape[1])
    t = _conv(t, jnp.transpose(jnp.flip(dec_w1, (2, 3)), (2, 3, 0, 1)),
              dec_b1, rows=32)
    for i in range(dec_res_w1.shape[0]):
        t = _conv(t, _w3x3(dec_res_w1[i]), dec_res_b1[i], rows=16,
                  relu_in=True, pw=dec_res_w2[i][:, :, 0, 0].T,
                  pb=dec_res_b2[i])
    t = _conv(t, _ctw(dec_w2), jnp.tile(dec_b2, 4), rows=16,
              relu_out=True)
    t = _s2d_inv(t)
    t = _conv(t, _ctw(dec_w3), jnp.tile(dec_b3, 4), rows=16)
    xr = _s2d_inv(t)
    return jnp.transpose(xr, (0, 3, 1, 2)), loss


# speedup vs baseline: 1.5773x; 1.5773x over previous
import functools

import jax
import jax.numpy as jnp
from jax import lax
from jax.experimental import pallas as pl
from jax.experimental.pallas import tpu as pltpu
from jax.experimental.pallas import tpu_sc as plsc

_INTERPRET = False


def _s2d(x):
    n, h, w, c = x.shape
    x = x.reshape(n, h // 2, 2, w // 2, 2, c)
    x = jnp.transpose(x, (0, 1, 3, 2, 4, 5))
    return x.reshape(n, h // 2, w // 2, 4 * c)


def _s2d_inv(x):
    n, h, w, c4 = x.shape
    c = c4 // 4
    x = x.reshape(n, h, w, 2, 2, c)
    x = jnp.transpose(x, (0, 1, 3, 2, 4, 5))
    return x.reshape(n, 2 * h, 2 * w, c)


_S2_TAP = ((-1, 1), (0, 0), (0, 1), (1, 0))


def _s2w(w):
    o, i = w.shape[0], w.shape[1]
    out = jnp.zeros((3, 3, 4 * i, o), jnp.float32)
    for ky in range(4):
        dy, py = _S2_TAP[ky]
        for kx in range(4):
            dx, px = _S2_TAP[kx]
            g = py * 2 + px
            out = out.at[dy + 1, dx + 1, g * i:(g + 1) * i, :].set(w[:, :, ky, kx].T)
    return out


_CT_TAP = {0: ((1, 0), (3, -1)), 1: ((0, 1), (2, 0))}


def _ctw(w):
    i, o = w.shape[0], w.shape[1]
    out = jnp.zeros((3, 3, i, 4 * o), jnp.float32)
    for py in (0, 1):
        for ky, dy in _CT_TAP[py]:
            for px in (0, 1):
                for kx, dx in _CT_TAP[px]:
                    g = py * 2 + px
                    out = out.at[dy + 1, dx + 1, :, g * o:(g + 1) * o].set(w[:, :, ky, kx])
    return out


def _w3x3(w):
    return jnp.transpose(w, (2, 3, 1, 0))


def _conv(x, w, b, rows, relu_in=False, relu_out=False, pw=None, pb=None):
    n, h, wd, c = x.shape
    co = w.shape[-1]
    cf = pw.shape[-1] if pw is not None else co
    r = rows
    nr = h // r
    xp = jnp.pad(x, ((0, 0), (1, 1), (1, 1), (0, 0)))
    xs = jnp.stack([xp[:, i * r:i * r + r + 2] for i in range(nr)], axis=1)
    w3 = w.reshape(3, 3 * c, co)
    bb = b.reshape(1, co)

    ins = [xs, w3, bb]
    in_specs = [
        pl.BlockSpec((1, 1, r + 2, wd + 2, c), lambda ni, ri: (ni, ri, 0, 0, 0)),
        pl.BlockSpec((3, 3 * c, co), lambda ni, ri: (0, 0, 0)),
        pl.BlockSpec((1, co), lambda ni, ri: (0, 0)),
    ]
    if pw is not None:
        ins += [pw, pb.reshape(1, cf)]
        in_specs += [
            pl.BlockSpec((co, cf), lambda ni, ri: (0, 0)),
            pl.BlockSpec((1, cf), lambda ni, ri: (0, 0)),
        ]

    def body(*refs):
        if pw is None:
            x_ref, w_ref, b_ref, o_ref = refs
        else:
            x_ref, w_ref, b_ref, pw_ref, pb_ref, o_ref = refs
        xb = x_ref[0, 0]
        if relu_in:
            xb = jnp.maximum(xb, 0.0)
        xc = jnp.concatenate(
            [xb[:, 0:wd], xb[:, 1:wd + 1], xb[:, 2:wd + 2]], axis=-1)
        acc = None
        for dy in range(3):
            t = jnp.dot(xc[dy:dy + r].reshape(r * wd, 3 * c), w_ref[dy],
                        preferred_element_type=jnp.float32)
            acc = t if acc is None else acc + t
        acc = acc + b_ref[:]
        if relu_out or pw is not None:
            acc = jnp.maximum(acc, 0.0)
        if pw is not None:
            acc = jnp.dot(acc, pw_ref[:],
                          preferred_element_type=jnp.float32) + pb_ref[:]
        o_ref[0, 0] = acc.reshape(r, wd, cf)

    out = pl.pallas_call(
        body,
        grid=(n, nr),
        in_specs=in_specs,
        out_specs=pl.BlockSpec((1, 1, r, wd, cf), lambda ni, ri: (ni, ri, 0, 0, 0)),
        out_shape=jax.ShapeDtypeStruct((n, nr, r, wd, cf), jnp.float32),
        interpret=_INTERPRET,
    )(*ins)
    return out.reshape(n, h, wd, cf)


def _vq(zf, embed, m):
    bsz, d = zf.shape
    e = embed.shape[0]
    nb = bsz // m
    et = embed.T

    def body(z_ref, et_ref, idx_ref, ls_ref):
        z = z_ref[:]
        ze = jnp.dot(z, et_ref[:], preferred_element_type=jnp.float32)
        z2 = jnp.sum(z * z, axis=1, keepdims=True)
        e2 = jnp.sum(et_ref[:] * et_ref[:], axis=0, keepdims=True)
        dist = z2 - 2.0 * ze + e2
        dmin = jnp.min(dist, axis=1, keepdims=True)
        ii = lax.broadcasted_iota(jnp.int32, dist.shape, 1)
        idxv = jnp.min(jnp.where(dist == dmin, ii, e), axis=1, keepdims=True)
        idx_ref[:] = idxv.astype(jnp.int32)
        s = jnp.sum(dmin).reshape(1, 1)

        @pl.when(pl.program_id(0) == 0)
        def _():
            ls_ref[:] = s

        @pl.when(pl.program_id(0) != 0)
        def _():
            ls_ref[:] = ls_ref[:] + s

    idx, ls = pl.pallas_call(
        body,
        grid=(nb,),
        in_specs=[
            pl.BlockSpec((m, d), lambda i: (i, 0)),
            pl.BlockSpec((d, e), lambda i: (0, 0)),
        ],
        out_specs=[
            pl.BlockSpec((m, 1), lambda i: (i, 0)),
            pl.BlockSpec((1, 1), lambda i: (0, 0)),
        ],
        out_shape=[
            jax.ShapeDtypeStruct((bsz, 1), jnp.int32),
            jax.ShapeDtypeStruct((1, 1), jnp.float32),
        ],
        interpret=_INTERPRET,
    )(zf, et)
    return idx.reshape(bsz), ls[0, 0]


def _gather_zq(embed, idx):
    e, d = embed.shape
    bsz = idx.shape[0]
    info = plsc.get_sparse_core_info()
    nc, ns = info.num_cores, info.num_subcores
    nw = nc * ns
    bpw = bsz // nw
    mesh = plsc.VectorSubcoreMesh(core_axis_name="c", subcore_axis_name="s")

    @functools.partial(
        pl.kernel,
        mesh=mesh,
        out_type=jax.ShapeDtypeStruct((bsz, d), jnp.float32),
        scratch_types=[
            pltpu.VMEM((bpw,), jnp.int32),
            pltpu.VMEM((bpw, d), jnp.float32),
            pltpu.SemaphoreType.DMA,
        ],
        compiler_params=pltpu.CompilerParams(use_tc_tiling_on_sc=False),
    )
    def k(table_hbm, idx_hbm, out_hbm, idx_v, rows_v, sem):
        wid = lax.axis_index("s") * nc + lax.axis_index("c")
        base = wid * bpw
        pltpu.sync_copy(idx_hbm.at[pl.ds(base, bpw)], idx_v)
        pltpu.async_copy(table_hbm.at[idx_v], rows_v, sem).wait()
        pltpu.sync_copy(rows_v, out_hbm.at[pl.ds(base, bpw)])

    return k(embed, idx)


def kernel(x, enc_w1, enc_b1, enc_w2, enc_b2, enc_w3, enc_b3, enc_res_w1,
           enc_res_b1, enc_res_w2, enc_res_b2, embed, dec_w1, dec_b1,
           dec_res_w1, dec_res_b1, dec_res_w2, dec_res_b2, dec_w2, dec_b2,
           dec_w3, dec_b3):
    n = x.shape[0]
    a = jnp.transpose(x, (0, 2, 3, 1))
    a = _s2d(a)
    a = _conv(a, _s2w(enc_w1), enc_b1, rows=16, relu_out=True)
    a = _s2d(a)
    a = _conv(a, _s2w(enc_w2), enc_b2, rows=8, relu_out=True)
    a = _conv(a, _w3x3(enc_w3), enc_b3, rows=16)
    for i in range(enc_res_w1.shape[0]):
        a = _conv(a, _w3x3(enc_res_w1[i]), enc_res_b1[i], rows=32,
                  relu_in=True, pw=enc_res_w2[i][:, :, 0, 0].T,
                  pb=enc_res_b2[i])

    zf = a.reshape(-1, embed.shape[1])
    idx, lsum = _vq(zf, embed, m=1024)
    loss = 1.25 * lsum / zf.size
    zq = _gather_zq(embed, idx)

    t = zq.reshape(n, 96, 96, embed.shape[1])
    t = _conv(t, jnp.transpose(jnp.flip(dec_w1, (2, 3)), (2, 3, 0, 1)),
              dec_b1, rows=32)
    for i in range(dec_res_w1.shape[0]):
        t = _conv(t, _w3x3(dec_res_w1[i]), dec_res_b1[i], rows=16,
                  relu_in=True, pw=dec_res_w2[i][:, :, 0, 0].T,
                  pb=dec_res_b2[i])
    t = _conv(t, _ctw(dec_w2), jnp.tile(dec_b2, 4), rows=16,
              relu_out=True)
    t = _s2d_inv(t)
    t = _conv(t, _ctw(dec_w3), jnp.tile(dec_b3, 4), rows=16)
    xr = _s2d_inv(t)
    return jnp.transpose(xr, (0, 3, 1, 2)), loss


# speedup vs baseline: 1.9064x; 1.2086x over previous
import functools

import jax
import jax.numpy as jnp
from jax import lax
from jax.experimental import pallas as pl
from jax.experimental.pallas import tpu as pltpu
from jax.experimental.pallas import tpu_sc as plsc

_INTERPRET = False


def _s2d(x):
    n, h, w, c = x.shape
    x = x.reshape(n, h // 2, 2, w // 2, 2, c)
    x = jnp.transpose(x, (0, 1, 3, 2, 4, 5))
    return x.reshape(n, h // 2, w // 2, 4 * c)


def _s2d_inv(x):
    n, h, w, c4 = x.shape
    c = c4 // 4
    x = x.reshape(n, h, w, 2, 2, c)
    x = jnp.transpose(x, (0, 1, 3, 2, 4, 5))
    return x.reshape(n, 2 * h, 2 * w, c)


_S2_TAP = ((-1, 1), (0, 0), (0, 1), (1, 0))


def _s2w(w):
    o, i = w.shape[0], w.shape[1]
    out = jnp.zeros((3, 3, 4 * i, o), jnp.float32)
    for ky in range(4):
        dy, py = _S2_TAP[ky]
        for kx in range(4):
            dx, px = _S2_TAP[kx]
            g = py * 2 + px
            out = out.at[dy + 1, dx + 1, g * i:(g + 1) * i, :].set(w[:, :, ky, kx].T)
    return out


_CT_TAP = {0: ((1, 0), (3, -1)), 1: ((0, 1), (2, 0))}


def _ctw(w):
    i, o = w.shape[0], w.shape[1]
    out = jnp.zeros((3, 3, i, 4 * o), jnp.float32)
    for py in (0, 1):
        for ky, dy in _CT_TAP[py]:
            for px in (0, 1):
                for kx, dx in _CT_TAP[px]:
                    g = py * 2 + px
                    out = out.at[dy + 1, dx + 1, :, g * o:(g + 1) * o].set(w[:, :, ky, kx])
    return out


def _w3x3(w):
    return jnp.transpose(w, (2, 3, 1, 0))


def _conv(x, w, b, rows, relu_in=False, relu_out=False, pw=None, pb=None):
    n, h, wd, c = x.shape
    co = w.shape[-1]
    cf = pw.shape[-1] if pw is not None else co
    r = rows
    nr = h // r
    xp = jnp.pad(x, ((0, 0), (1, 1), (1, 1), (0, 0)))
    xs = jnp.stack([xp[:, i * r:i * r + r + 2] for i in range(nr)], axis=1)
    w3 = w.reshape(3, 3 * c, co)
    bb = b.reshape(1, co)

    ins = [xs, w3, bb]
    in_specs = [
        pl.BlockSpec((1, 1, r + 2, wd + 2, c), lambda ni, ri: (ni, ri, 0, 0, 0)),
        pl.BlockSpec((3, 3 * c, co), lambda ni, ri: (0, 0, 0)),
        pl.BlockSpec((1, co), lambda ni, ri: (0, 0)),
    ]
    if pw is not None:
        ins += [pw, pb.reshape(1, cf)]
        in_specs += [
            pl.BlockSpec((co, cf), lambda ni, ri: (0, 0)),
            pl.BlockSpec((1, cf), lambda ni, ri: (0, 0)),
        ]

    def body(*refs):
        if pw is None:
            x_ref, w_ref, b_ref, o_ref = refs
        else:
            x_ref, w_ref, b_ref, pw_ref, pb_ref, o_ref = refs
        xb = x_ref[0, 0]
        if relu_in:
            xb = jnp.maximum(xb, 0.0)
        xc = jnp.concatenate(
            [xb[:, 0:wd], xb[:, 1:wd + 1], xb[:, 2:wd + 2]], axis=-1)
        acc = None
        for dy in range(3):
            t = jnp.dot(xc[dy:dy + r].reshape(r * wd, 3 * c), w_ref[dy],
                        preferred_element_type=jnp.float32)
            acc = t if acc is None else acc + t
        acc = acc + b_ref[:]
        if relu_out or pw is not None:
            acc = jnp.maximum(acc, 0.0)
        if pw is not None:
            acc = jnp.dot(acc, pw_ref[:],
                          preferred_element_type=jnp.float32) + pb_ref[:]
        o_ref[0, 0] = acc.reshape(r, wd, cf)

    out = pl.pallas_call(
        body,
        grid=(n, nr),
        in_specs=in_specs,
        out_specs=pl.BlockSpec((1, 1, r, wd, cf), lambda ni, ri: (ni, ri, 0, 0, 0)),
        out_shape=jax.ShapeDtypeStruct((n, nr, r, wd, cf), jnp.float32),
        interpret=_INTERPRET,
    )(*ins)
    return out.reshape(n, h, wd, cf)


def _vq(zf, embed, m):
    bsz, d = zf.shape
    e = embed.shape[0]
    nb = bsz // m
    et = embed.T

    def body(z_ref, et_ref, idx_ref, ls_ref):
        z = z_ref[:]
        ze = jnp.dot(z, et_ref[:], preferred_element_type=jnp.float32)
        z2 = jnp.sum(z * z, axis=1, keepdims=True)
        e2 = jnp.sum(et_ref[:] * et_ref[:], axis=0, keepdims=True)
        dist = z2 - 2.0 * ze + e2
        dmin = jnp.min(dist, axis=1, keepdims=True)
        ii = lax.broadcasted_iota(jnp.int32, dist.shape, 1)
        idxv = jnp.min(jnp.where(dist == dmin, ii, e), axis=1, keepdims=True)
        idx_ref[:] = idxv.astype(jnp.int32)
        s = jnp.sum(dmin).reshape(1, 1)

        @pl.when(pl.program_id(0) == 0)
        def _():
            ls_ref[:] = s

        @pl.when(pl.program_id(0) != 0)
        def _():
            ls_ref[:] = ls_ref[:] + s

    idx, ls = pl.pallas_call(
        body,
        grid=(nb,),
        in_specs=[
            pl.BlockSpec((m, d), lambda i: (i, 0)),
            pl.BlockSpec((d, e), lambda i: (0, 0)),
        ],
        out_specs=[
            pl.BlockSpec((m, 1), lambda i: (i, 0)),
            pl.BlockSpec((1, 1), lambda i: (0, 0)),
        ],
        out_shape=[
            jax.ShapeDtypeStruct((bsz, 1), jnp.int32),
            jax.ShapeDtypeStruct((1, 1), jnp.float32),
        ],
        interpret=_INTERPRET,
    )(zf, et)
    return idx.reshape(bsz), ls[0, 0]


def _gather_zq(embed, idx):
    e, d = embed.shape
    bsz = idx.shape[0]
    info = plsc.get_sparse_core_info()
    nc, ns = info.num_cores, info.num_subcores
    nw = nc * ns
    bpw = bsz // nw
    mesh = plsc.VectorSubcoreMesh(core_axis_name="c", subcore_axis_name="s")

    @functools.partial(
        pl.kernel,
        mesh=mesh,
        out_type=jax.ShapeDtypeStruct((bsz, d), jnp.float32),
        scratch_types=[
            pltpu.VMEM((bpw,), jnp.int32),
            pltpu.VMEM((bpw, d), jnp.float32),
            pltpu.VMEM((e, d), jnp.float32),
            pltpu.SemaphoreType.DMA,
        ],
        compiler_params=pltpu.CompilerParams(
            use_tc_tiling_on_sc=False, needs_layout_passes=False),
    )
    def k(table_hbm, idx_hbm, out_hbm, idx_v, rows_v, tbl_v, sem):
        wid = lax.axis_index("s") * nc + lax.axis_index("c")
        base = wid * bpw
        pltpu.sync_copy(table_hbm, tbl_v)
        pltpu.sync_copy(idx_hbm.at[pl.ds(base, bpw)], idx_v)
        lanes = 16
        ngroup = bpw // lanes

        def group(g, carry):
            iv = idx_v[pl.ds(g * lanes, lanes)]
            riota = g * lanes + lax.iota(jnp.int32, lanes)
            for c in range(d):
                cc = jnp.full((lanes,), c, jnp.int32)
                v = plsc.load_gather(tbl_v, [iv, cc])
                plsc.store_scatter(rows_v, [riota, cc], v)
            return carry

        lax.fori_loop(0, ngroup, group, 0)
        pltpu.sync_copy(rows_v, out_hbm.at[pl.ds(base, bpw)])

    return k(embed, idx)


def kernel(x, enc_w1, enc_b1, enc_w2, enc_b2, enc_w3, enc_b3, enc_res_w1,
           enc_res_b1, enc_res_w2, enc_res_b2, embed, dec_w1, dec_b1,
           dec_res_w1, dec_res_b1, dec_res_w2, dec_res_b2, dec_w2, dec_b2,
           dec_w3, dec_b3):
    n = x.shape[0]
    a = jnp.transpose(x, (0, 2, 3, 1))
    a = _s2d(a)
    a = _conv(a, _s2w(enc_w1), enc_b1, rows=16, relu_out=True)
    a = _s2d(a)
    a = _conv(a, _s2w(enc_w2), enc_b2, rows=8, relu_out=True)
    a = _conv(a, _w3x3(enc_w3), enc_b3, rows=16)
    for i in range(enc_res_w1.shape[0]):
        a = _conv(a, _w3x3(enc_res_w1[i]), enc_res_b1[i], rows=32,
                  relu_in=True, pw=enc_res_w2[i][:, :, 0, 0].T,
                  pb=enc_res_b2[i])

    zf = a.reshape(-1, embed.shape[1])
    idx, lsum = _vq(zf, embed, m=1024)
    loss = 1.25 * lsum / zf.size
    zq = _gather_zq(embed, idx)

    t = zq.reshape(n, 96, 96, embed.shape[1])
    t = _conv(t, jnp.transpose(jnp.flip(dec_w1, (2, 3)), (2, 3, 0, 1)),
              dec_b1, rows=32)
    for i in range(dec_res_w1.shape[0]):
        t = _conv(t, _w3x3(dec_res_w1[i]), dec_res_b1[i], rows=16,
                  relu_in=True, pw=dec_res_w2[i][:, :, 0, 0].T,
                  pb=dec_res_b2[i])
    t = _conv(t, _ctw(dec_w2), jnp.tile(dec_b2, 4), rows=16,
              relu_out=True)
    t = _s2d_inv(t)
    t = _conv(t, _ctw(dec_w3), jnp.tile(dec_b3, 4), rows=16)
    xr = _s2d_inv(t)
    return jnp.transpose(xr, (0, 3, 1, 2)), loss


# speedup vs baseline: 1.9248x; 1.0097x over previous
import functools

import jax
import jax.numpy as jnp
from jax import lax
from jax.experimental import pallas as pl
from jax.experimental.pallas import tpu as pltpu
from jax.experimental.pallas import tpu_sc as plsc

_INTERPRET = False


def _s2d(x):
    n, h, w, c = x.shape
    x = x.reshape(n, h // 2, 2, w // 2, 2, c)
    x = jnp.transpose(x, (0, 1, 3, 2, 4, 5))
    return x.reshape(n, h // 2, w // 2, 4 * c)


def _s2d_inv(x):
    n, h, w, c4 = x.shape
    c = c4 // 4
    x = x.reshape(n, h, w, 2, 2, c)
    x = jnp.transpose(x, (0, 1, 3, 2, 4, 5))
    return x.reshape(n, 2 * h, 2 * w, c)


_S2_TAP = ((-1, 1), (0, 0), (0, 1), (1, 0))


def _s2w(w):
    o, i = w.shape[0], w.shape[1]
    out = jnp.zeros((3, 3, 4 * i, o), jnp.float32)
    for ky in range(4):
        dy, py = _S2_TAP[ky]
        for kx in range(4):
            dx, px = _S2_TAP[kx]
            g = py * 2 + px
            out = out.at[dy + 1, dx + 1, g * i:(g + 1) * i, :].set(w[:, :, ky, kx].T)
    return out


_CT_TAP = {0: ((1, 0), (3, -1)), 1: ((0, 1), (2, 0))}


def _ctw(w):
    i, o = w.shape[0], w.shape[1]
    out = jnp.zeros((3, 3, i, 4 * o), jnp.float32)
    for py in (0, 1):
        for ky, dy in _CT_TAP[py]:
            for px in (0, 1):
                for kx, dx in _CT_TAP[px]:
                    g = py * 2 + px
                    out = out.at[dy + 1, dx + 1, :, g * o:(g + 1) * o].set(w[:, :, ky, kx])
    return out


def _w3x3(w):
    return jnp.transpose(w, (2, 3, 1, 0))


def _conv(x, w, b, rows, relu_in=False, relu_out=False, pw=None, pb=None):
    n, h, wd, c = x.shape
    co = w.shape[-1]
    cf = pw.shape[-1] if pw is not None else co
    r = rows
    nr = h // r
    xp = jnp.pad(x, ((0, 0), (1, 1), (1, 1), (0, 0)))
    xs = jnp.stack([xp[:, i * r:i * r + r + 2] for i in range(nr)], axis=1)
    w3 = w.reshape(3, 3 * c, co)
    bb = b.reshape(1, co)

    ins = [xs, w3, bb]
    in_specs = [
        pl.BlockSpec((1, 1, r + 2, wd + 2, c), lambda ni, ri: (ni, ri, 0, 0, 0)),
        pl.BlockSpec((3, 3 * c, co), lambda ni, ri: (0, 0, 0)),
        pl.BlockSpec((1, co), lambda ni, ri: (0, 0)),
    ]
    if pw is not None:
        ins += [pw, pb.reshape(1, cf)]
        in_specs += [
            pl.BlockSpec((co, cf), lambda ni, ri: (0, 0)),
            pl.BlockSpec((1, cf), lambda ni, ri: (0, 0)),
        ]

    def body(*refs):
        if pw is None:
            x_ref, w_ref, b_ref, o_ref = refs
        else:
            x_ref, w_ref, b_ref, pw_ref, pb_ref, o_ref = refs
        xb = x_ref[0, 0]
        if relu_in:
            xb = jnp.maximum(xb, 0.0)
        xb = xb.astype(jnp.bfloat16)
        xc = jnp.concatenate(
            [xb[:, 0:wd], xb[:, 1:wd + 1], xb[:, 2:wd + 2]], axis=-1)
        acc = None
        for dy in range(3):
            t = jnp.dot(xc[dy:dy + r].reshape(r * wd, 3 * c),
                        w_ref[dy].astype(jnp.bfloat16),
                        preferred_element_type=jnp.float32)
            acc = t if acc is None else acc + t
        acc = acc + b_ref[:]
        if relu_out or pw is not None:
            acc = jnp.maximum(acc, 0.0)
        if pw is not None:
            acc = jnp.dot(acc.astype(jnp.bfloat16),
                          pw_ref[:].astype(jnp.bfloat16),
                          preferred_element_type=jnp.float32) + pb_ref[:]
        o_ref[0, 0] = acc.reshape(r, wd, cf)

    out = pl.pallas_call(
        body,
        grid=(n, nr),
        in_specs=in_specs,
        out_specs=pl.BlockSpec((1, 1, r, wd, cf), lambda ni, ri: (ni, ri, 0, 0, 0)),
        out_shape=jax.ShapeDtypeStruct((n, nr, r, wd, cf), jnp.float32),
        interpret=_INTERPRET,
    )(*ins)
    return out.reshape(n, h, wd, cf)


def _vq(zf, embed, m):
    bsz, d = zf.shape
    e = embed.shape[0]
    nb = bsz // m
    et = embed.T

    def body(z_ref, et_ref, idx_ref, ls_ref):
        z = z_ref[:]
        ze = jnp.dot(z, et_ref[:], preferred_element_type=jnp.float32)
        z2 = jnp.sum(z * z, axis=1, keepdims=True)
        e2 = jnp.sum(et_ref[:] * et_ref[:], axis=0, keepdims=True)
        dist = z2 - 2.0 * ze + e2
        dmin = jnp.min(dist, axis=1, keepdims=True)
        ii = lax.broadcasted_iota(jnp.int32, dist.shape, 1)
        idxv = jnp.min(jnp.where(dist == dmin, ii, e), axis=1, keepdims=True)
        idx_ref[:] = idxv.astype(jnp.int32)
        s = jnp.sum(dmin).reshape(1, 1)

        @pl.when(pl.program_id(0) == 0)
        def _():
            ls_ref[:] = s

        @pl.when(pl.program_id(0) != 0)
        def _():
            ls_ref[:] = ls_ref[:] + s

    idx, ls = pl.pallas_call(
        body,
        grid=(nb,),
        in_specs=[
            pl.BlockSpec((m, d), lambda i: (i, 0)),
            pl.BlockSpec((d, e), lambda i: (0, 0)),
        ],
        out_specs=[
            pl.BlockSpec((m, 1), lambda i: (i, 0)),
            pl.BlockSpec((1, 1), lambda i: (0, 0)),
        ],
        out_shape=[
            jax.ShapeDtypeStruct((bsz, 1), jnp.int32),
            jax.ShapeDtypeStruct((1, 1), jnp.float32),
        ],
        interpret=_INTERPRET,
    )(zf, et)
    return idx.reshape(bsz), ls[0, 0]


def _gather_zq(embed, idx):
    e, d = embed.shape
    bsz = idx.shape[0]
    info = plsc.get_sparse_core_info()
    nc, ns = info.num_cores, info.num_subcores
    nw = nc * ns
    bpw = bsz // nw
    mesh = plsc.VectorSubcoreMesh(core_axis_name="c", subcore_axis_name="s")

    @functools.partial(
        pl.kernel,
        mesh=mesh,
        out_type=jax.ShapeDtypeStruct((bsz, d), jnp.float32),
        scratch_types=[
            pltpu.VMEM((bpw,), jnp.int32),
            pltpu.VMEM((bpw, d), jnp.float32),
            pltpu.VMEM((e, d), jnp.float32),
            pltpu.SemaphoreType.DMA,
        ],
        compiler_params=pltpu.CompilerParams(
            use_tc_tiling_on_sc=False, needs_layout_passes=False),
    )
    def k(table_hbm, idx_hbm, out_hbm, idx_v, rows_v, tbl_v, sem):
        wid = lax.axis_index("s") * nc + lax.axis_index("c")
        base = wid * bpw
        pltpu.sync_copy(table_hbm, tbl_v)
        pltpu.sync_copy(idx_hbm.at[pl.ds(base, bpw)], idx_v)
        lanes = 16
        ngroup = bpw // lanes

        def group(g, carry):
            iv = idx_v[pl.ds(g * lanes, lanes)]
            riota = g * lanes + lax.iota(jnp.int32, lanes)
            for c in range(d):
                cc = jnp.full((lanes,), c, jnp.int32)
                v = plsc.load_gather(tbl_v, [iv, cc])
                plsc.store_scatter(rows_v, [riota, cc], v)
            return carry

        lax.fori_loop(0, ngroup, group, 0)
        pltpu.sync_copy(rows_v, out_hbm.at[pl.ds(base, bpw)])

    return k(embed, idx)


def kernel(x, enc_w1, enc_b1, enc_w2, enc_b2, enc_w3, enc_b3, enc_res_w1,
           enc_res_b1, enc_res_w2, enc_res_b2, embed, dec_w1, dec_b1,
           dec_res_w1, dec_res_b1, dec_res_w2, dec_res_b2, dec_w2, dec_b2,
           dec_w3, dec_b3):
    n = x.shape[0]
    a = jnp.transpose(x, (0, 2, 3, 1))
    a = _s2d(a)
    a = _conv(a, _s2w(enc_w1), enc_b1, rows=16, relu_out=True)
    a = _s2d(a)
    a = _conv(a, _s2w(enc_w2), enc_b2, rows=8, relu_out=True)
    a = _conv(a, _w3x3(enc_w3), enc_b3, rows=16)
    for i in range(enc_res_w1.shape[0]):
        a = _conv(a, _w3x3(enc_res_w1[i]), enc_res_b1[i], rows=32,
                  relu_in=True, pw=enc_res_w2[i][:, :, 0, 0].T,
                  pb=enc_res_b2[i])

    zf = a.reshape(-1, embed.shape[1])
    idx, lsum = _vq(zf, embed, m=1024)
    loss = 1.25 * lsum / zf.size
    zq = _gather_zq(embed, idx)

    t = zq.reshape(n, 96, 96, embed.shape[1])
    t = _conv(t, jnp.transpose(jnp.flip(dec_w1, (2, 3)), (2, 3, 0, 1)),
              dec_b1, rows=32)
    for i in range(dec_res_w1.shape[0]):
        t = _conv(t, _w3x3(dec_res_w1[i]), dec_res_b1[i], rows=16,
                  relu_in=True, pw=dec_res_w2[i][:, :, 0, 0].T,
                  pb=dec_res_b2[i])
    t = _conv(t, _ctw(dec_w2), jnp.tile(dec_b2, 4), rows=16,
              relu_out=True)
    t = _s2d_inv(t)
    t = _conv(t, _ctw(dec_w3), jnp.tile(dec_b3, 4), rows=16)
    xr = _s2d_inv(t)
    return jnp.transpose(xr, (0, 3, 1, 2)), loss
